# baseline (speedup 1.0000x reference)
import functools
import jax
import jax.numpy as jnp
from jax import lax
from jax.experimental import pallas as pl
from jax.experimental.pallas import tpu as pltpu
from jax.experimental.pallas import tpu_sc as plsc

_MAXR = 32
_HEADS = 16
_S = 2048
_NW = 32
_RPW = (_HEADS * _S) // _NW
_ML = 4112


@functools.partial(
    pl.kernel,
    out_type=jax.ShapeDtypeStruct((_HEADS * _S * _S,), jnp.float32),
    mesh=plsc.VectorSubcoreMesh(core_axis_name="c", subcore_axis_name="s"),
    scratch_types=[
        pltpu.VMEM((_HEADS * 64,), jnp.float32),
        pltpu.VMEM((8 * _ML,), jnp.float32),
        pltpu.SemaphoreType.DMA,
    ],
)
def _sc_bias(tab_hbm, out_hbm, tab_v, m8_v, sem):
    wid = lax.axis_index("s") * 2 + lax.axis_index("c")
    h = wid // 2
    pltpu.sync_copy(tab_hbm, tab_v)

    rchunks = [tab_v[pl.ds(pl.multiple_of(h * 64 + 16 * kc, 16), 16)] for kc in range(4)]
    g62 = jnp.full((16,), rchunks[0][0], jnp.float32)
    g0 = jnp.full((16,), rchunks[3][14], jnp.float32)

    for p in range(8):

        def fbody(c, _, p=p):
            off = pl.multiple_of(p * _ML + c * 16, 16)
            m8_v[pl.ds(off, 16)] = jnp.where(c * 16 < 2048, g62, g0)
            return 0

        lax.fori_loop(0, _ML // 16, fbody, 0)

    for p in range(8):
        for kc in range(4):
            m8_v[pl.ds(p * _ML + 2016 - p + 16 * kc, 16)] = rchunks[kc]

    base = wid * _RPW

    def rbody(it, _):
        r0 = base + it * 8
        i0 = lax.rem(r0, _S)
        s0 = 2040 - i0
        copies = []
        for j in range(8):
            src_off = pl.multiple_of((7 - j) * _ML + s0, 8)
            dst_off = pl.multiple_of((r0 + j) * _S, 8)
            copies.append(
                pltpu.async_copy(
                    m8_v.at[pl.ds(src_off, _S)],
                    out_hbm.at[pl.ds(dst_off, _S)],
                    sem,
                )
            )
        for c in copies:
            c.wait()
        return 0

    lax.fori_loop(0, _RPW // 8, rbody, 0)


def kernel(seq_len, table):
    rtab = jnp.concatenate([table[::-1], table[:1]], axis=0).T
    out = _sc_bias(rtab.reshape(-1))
    return out.reshape(1, _HEADS, _S, _S)


# speedup vs baseline: 1.0062x; 1.0062x over previous
import functools
import jax
import jax.numpy as jnp
from jax import lax
from jax.experimental import pallas as pl
from jax.experimental.pallas import tpu as pltpu
from jax.experimental.pallas import tpu_sc as plsc

_MAXR = 32
_HEADS = 16
_S = 2048
_NW = 32
_RPW = (_HEADS * _S) // _NW
_ML = 4112


@functools.partial(
    pl.kernel,
    out_type=jax.ShapeDtypeStruct((_HEADS * _S * _S,), jnp.float32),
    mesh=plsc.VectorSubcoreMesh(core_axis_name="c", subcore_axis_name="s"),
    scratch_types=[
        pltpu.VMEM((_HEADS * 64,), jnp.float32),
        pltpu.VMEM((8 * _ML,), jnp.float32),
        pltpu.SemaphoreType.DMA,
    ],
)
def _sc_bias(tab_hbm, out_hbm, tab_v, m8_v, sem):
    wid = lax.axis_index("s") * 2 + lax.axis_index("c")
    h = wid // 2
    pltpu.sync_copy(tab_hbm, tab_v)

    rchunks = [tab_v[pl.ds(pl.multiple_of(h * 64 + 16 * kc, 16), 16)] for kc in range(4)]
    g62 = jnp.full((16,), rchunks[0][0], jnp.float32)
    g0 = jnp.full((16,), rchunks[3][14], jnp.float32)

    for p in range(8):

        def fbody(c, _, p=p):
            off = pl.multiple_of(p * _ML + c * 16, 16)
            m8_v[pl.ds(off, 16)] = jnp.where(c * 16 < 2048, g62, g0)
            return 0

        lax.fori_loop(0, _ML // 16, fbody, 0)

    for p in range(8):
        for kc in range(4):
            m8_v[pl.ds(p * _ML + 2016 - p + 16 * kc, 16)] = rchunks[kc]

    base = wid * _RPW

    def rbody(it, _):
        r0 = base + it * 16
        i0 = lax.rem(r0, _S)
        copies = []
        for g in range(2):
            s0 = 2040 - (i0 + 8 * g)
            for j in range(8):
                src_off = pl.multiple_of((7 - j) * _ML + s0, 8)
                dst_off = pl.multiple_of((r0 + 8 * g + j) * _S, 8)
                copies.append(
                    pltpu.async_copy(
                        m8_v.at[pl.ds(src_off, _S)],
                        out_hbm.at[pl.ds(dst_off, _S)],
                        sem,
                    )
                )
        for c in copies:
            c.wait()
        return 0

    lax.fori_loop(0, _RPW // 16, rbody, 0)


def kernel(seq_len, table):
    rtab = jnp.concatenate([table[::-1], table[:1]], axis=0).T
    out = _sc_bias(rtab.reshape(-1))
    return out.reshape(1, _HEADS, _S, _S)


# speedup vs baseline: 3.9426x; 3.9184x over previous
import jax
import jax.numpy as jnp
from jax.experimental import pallas as pl
from jax.experimental.pallas import tpu as pltpu

_MAXR = 32
_HEADS = 16
_S = 2048
_BI = 512
_W = 256
_NCH = 4


def _bias_kernel(tab_ref, out_ref, pat_ref):
    s = pl.program_id(1)

    @pl.when(s == 0)
    def _build_pattern():
        ij = jax.lax.broadcasted_iota(jnp.int32, (_BI, _W), 0) - jax.lax.broadcasted_iota(
            jnp.int32, (_BI, _W), 1
        )
        lut2 = jnp.broadcast_to(tab_ref[0, 0, :], (_BI, 128))
        for k in range(_NCH):
            rp = jnp.clip(ij + (_W - _W * k), -_MAXR + 1, _MAXR - 1) + (_MAXR - 1)
            pat_ref[k, :, :] = jnp.take_along_axis(lut2, rp, axis=1)

    c_lo = jnp.full((_BI, _W), tab_ref[0, 0, 2 * _MAXR - 2], jnp.float32)
    c_hi = jnp.full((_BI, _W), tab_ref[0, 0, 0], jnp.float32)
    s2 = s * 2
    for c in range(_S // _W):
        @pl.when(s2 - c >= 2)
        def _lo_const():
            out_ref[0, :, c * _W : (c + 1) * _W] = c_lo

        @pl.when(s2 - c <= -3)
        def _hi_const():
            out_ref[0, :, c * _W : (c + 1) * _W] = c_hi

        @pl.when(jnp.logical_and(s2 - c >= -2, s2 - c <= 1))
        def _band():
            out_ref[0, :, c * _W : (c + 1) * _W] = pat_ref[c - s2 + 1, :, :]


def kernel(seq_len, table):
    tab = jnp.zeros((_HEADS, 1, 128), jnp.float32)
    tab = tab.at[:, 0, : 2 * _MAXR - 1].set(table.T)
    out = pl.pallas_call(
        _bias_kernel,
        grid=(_HEADS, _S // _BI),
        in_specs=[pl.BlockSpec((1, 1, 128), lambda h, s: (h, 0, 0))],
        out_specs=pl.BlockSpec((1, _BI, _S), lambda h, s: (h, s, 0)),
        out_shape=jax.ShapeDtypeStruct((_HEADS, _S, _S), jnp.float32),
        scratch_shapes=[pltpu.VMEM((_NCH, _BI, _W), jnp.float32)],
    )(tab)
    return out[None]
